# baseline (speedup 1.0000x reference)
import functools

import jax
import jax.numpy as jnp
from jax import lax
from jax.experimental import pallas as pl
from jax.experimental.pallas import tpu as pltpu
from jax.experimental.pallas import tpu_sc as plsc

N = 10000
E = 320000
D_IN = 128
D_HID = 128
D_OUT = 64

NC = 2
NS = 16
NW = NC * NS
EPW = E // NW
CHUNK = 128
NCH = E // CHUNK
RPT = 624
TAIL0 = NS * RPT
TAILN = N - TAIL0

_MESH = dict(core_axis_name="c", subcore_axis_name="s")


def _deg_body(col_hbm, out_hbm, idxbuf, acc):
    c = lax.axis_index("c")
    s = lax.axis_index("s")
    w = s * NC + c

    def zero(i, carry):
        acc[pl.ds(i * 16, 16)] = jnp.zeros((16,), jnp.float32)
        return carry

    lax.fori_loop(0, N // 16, zero, 0)

    pltpu.sync_copy(col_hbm.at[pl.ds(w * EPW, EPW)], idxbuf)
    ones = jnp.ones((16,), jnp.float32)

    def body(i, carry):
        idx = idxbuf[pl.ds(i * 16, 16)]
        plsc.addupdate_scatter(acc, [idx], ones)
        return carry

    lax.fori_loop(0, EPW // 16, body, 0)
    pltpu.sync_copy(acc, out_hbm.at[w])


_SC_PARAMS = pltpu.CompilerParams(needs_layout_passes=False)
_SC_PARAMS_LINEAR = pltpu.CompilerParams(
    needs_layout_passes=False, use_tc_tiling_on_sc=False)

_deg_call = pl.kernel(
    _deg_body,
    out_type=jax.ShapeDtypeStruct((NW, N), jnp.float32),
    mesh=plsc.VectorSubcoreMesh(**_MESH),
    compiler_params=_SC_PARAMS,
    scratch_types=[
        pltpu.VMEM((EPW,), jnp.int32),
        pltpu.VMEM((N,), jnp.float32),
    ],
)


TPW = NCH // NS
TREM = NCH % NS
MAXT = TPW + 1
NBUF = 4


def _agg_body(table_hbm, rowc_hbm, colc_hbm, zeros_hbm, out_hbm,
              accum, idxr, idxc, rows, gsems, ssems):
    c = lax.axis_index("c")
    s = lax.axis_index("s")

    off = pl.multiple_of(s * RPT, 8)
    pltpu.sync_copy(zeros_hbm.at[pl.ds(off, RPT)], accum.at[pl.ds(off, RPT)])

    @pl.when(s == NS - 1)
    def _():
        pltpu.sync_copy(zeros_hbm.at[pl.ds(TAIL0, TAILN)],
                        accum.at[pl.ds(TAIL0, TAILN)])

    extra = jnp.minimum(s, TREM)
    gstart = s * TPW + extra
    nt = jnp.where(s < TREM, MAXT, TPW)

    @pl.when(s < TREM)
    def _():
        pltpu.sync_copy(rowc_hbm.at[pl.ds(gstart, MAXT)], idxr)
        pltpu.sync_copy(colc_hbm.at[pl.ds(gstart, MAXT)], idxc)

    @pl.when(s >= TREM)
    def _():
        pltpu.sync_copy(rowc_hbm.at[pl.ds(gstart, TPW)],
                        idxr.at[pl.ds(0, TPW)])
        pltpu.sync_copy(colc_hbm.at[pl.ds(gstart, TPW)],
                        idxc.at[pl.ds(0, TPW)])

    plsc.subcore_barrier()
    table_c = table_hbm.at[c]

    for b in range(NBUF):
        pltpu.async_copy(table_c.at[idxr.at[b]], rows.at[b], gsems.at[b])

    def outer(g, carry):
        for b in range(NBUF):
            t = g * NBUF + b

            @pl.when(t < nt)
            def _():
                pltpu.make_async_copy(
                    table_c.at[idxr.at[b]], rows.at[b], gsems.at[b]).wait()
                pltpu.async_copy(rows.at[b], accum.at[idxc.at[t]],
                                 ssems.at[b], add=True)

        for b in range(NBUF):
            t = g * NBUF + b

            @pl.when(t < nt)
            def _():
                pltpu.make_async_copy(rows.at[b], accum.at[idxc.at[t]],
                                      ssems.at[b]).wait()

                @pl.when(t + NBUF < nt)
                def _():
                    pltpu.async_copy(table_c.at[idxr.at[t + NBUF]],
                                     rows.at[b], gsems.at[b])

        return carry

    lax.fori_loop(0, (MAXT + NBUF - 1) // NBUF, outer, 0)
    plsc.subcore_barrier()
    pltpu.sync_copy(accum.at[pl.ds(off, RPT)], out_hbm.at[c, pl.ds(off, RPT)])

    @pl.when(s == NS - 1)
    def _():
        pltpu.sync_copy(accum.at[pl.ds(TAIL0, TAILN)],
                        out_hbm.at[c, pl.ds(TAIL0, TAILN)])


def _make_agg(d):
    return pl.kernel(
        _agg_body,
        out_type=jax.ShapeDtypeStruct((NC, N, d), jnp.float32),
        mesh=plsc.VectorSubcoreMesh(**_MESH),
        compiler_params=_SC_PARAMS_LINEAR,
        scratch_types=[
            pltpu.VMEM_SHARED((N, d), jnp.float32),
            pltpu.VMEM((MAXT, CHUNK), jnp.int32),
            pltpu.VMEM((MAXT, CHUNK), jnp.int32),
            pltpu.VMEM((NBUF, CHUNK, d), jnp.float32),
            pltpu.SemaphoreType.DMA((NBUF,)),
            pltpu.SemaphoreType.DMA((NBUF,)),
        ],
    )


_agg_hid = _make_agg(D_HID // NC)
_agg_out = _make_agg(D_OUT // NC)


_BN = N


def _tc1_body(x_ref, w1i_ref, w1r_ref, b1_ref, degp_ref,
              h0s_ref, root1_ref, dinv_ref):
    x = x_ref[...]
    deg = jnp.sum(degp_ref[...], axis=0)
    dinv = jnp.where(deg > 0, lax.rsqrt(jnp.maximum(deg, 1.0)), 0.0)
    h0 = jnp.dot(x, w1i_ref[...], preferred_element_type=jnp.float32)
    h0s = h0 * dinv[:, None]
    h0s_ref[0] = h0s[:, : D_HID // NC]
    h0s_ref[1] = h0s[:, D_HID // NC :]
    root1_ref[...] = (
        jnp.dot(x, w1r_ref[...], preferred_element_type=jnp.float32)
        + b1_ref[...][None, :]
    )
    dinv_ref[...] = dinv


def _tc1_call(x, w1i, w1r, b1, deg_parts):
    return pl.pallas_call(
        _tc1_body,
        grid=(pl.cdiv(N, _BN),),
        in_specs=[
            pl.BlockSpec((_BN, D_IN), lambda i: (i, 0)),
            pl.BlockSpec((D_IN, D_HID), lambda i: (0, 0)),
            pl.BlockSpec((D_IN, D_HID), lambda i: (0, 0)),
            pl.BlockSpec((D_HID,), lambda i: (0,)),
            pl.BlockSpec((NW, _BN), lambda i: (0, i)),
        ],
        out_specs=[
            pl.BlockSpec((NC, _BN, D_HID // NC), lambda i: (0, i, 0)),
            pl.BlockSpec((_BN, D_HID), lambda i: (i, 0)),
            pl.BlockSpec((_BN,), lambda i: (i,)),
        ],
        out_shape=[
            jax.ShapeDtypeStruct((NC, N, D_HID // NC), jnp.float32),
            jax.ShapeDtypeStruct((N, D_HID), jnp.float32),
            jax.ShapeDtypeStruct((N,), jnp.float32),
        ],
    )(x, w1i, w1r, b1, deg_parts)


def _tc2_body(agg_ref, root1_ref, dinv_ref, w2i_ref, w2r_ref, b2_ref,
              h1s_ref, root2_ref):
    agg = jnp.concatenate([agg_ref[0], agg_ref[1]], axis=-1)
    dinv = dinv_ref[...]
    out1 = jnp.maximum(agg * dinv[:, None] + root1_ref[...], 0.0)
    h1 = jnp.dot(out1, w2i_ref[...], preferred_element_type=jnp.float32)
    h1s = h1 * dinv[:, None]
    h1s_ref[0] = h1s[:, : D_OUT // NC]
    h1s_ref[1] = h1s[:, D_OUT // NC :]
    root2_ref[...] = (
        jnp.dot(out1, w2r_ref[...], preferred_element_type=jnp.float32)
        + b2_ref[...][None, :]
    )


def _tc2_call(agg, root1, dinv, w2i, w2r, b2):
    return pl.pallas_call(
        _tc2_body,
        grid=(pl.cdiv(N, _BN),),
        in_specs=[
            pl.BlockSpec((NC, _BN, D_HID // NC), lambda i: (0, i, 0)),
            pl.BlockSpec((_BN, D_HID), lambda i: (i, 0)),
            pl.BlockSpec((_BN,), lambda i: (i,)),
            pl.BlockSpec((D_HID, D_OUT), lambda i: (0, 0)),
            pl.BlockSpec((D_HID, D_OUT), lambda i: (0, 0)),
            pl.BlockSpec((D_OUT,), lambda i: (0,)),
        ],
        out_specs=[
            pl.BlockSpec((NC, _BN, D_OUT // NC), lambda i: (0, i, 0)),
            pl.BlockSpec((_BN, D_OUT), lambda i: (i, 0)),
        ],
        out_shape=[
            jax.ShapeDtypeStruct((NC, N, D_OUT // NC), jnp.float32),
            jax.ShapeDtypeStruct((N, D_OUT), jnp.float32),
        ],
    )(agg, root1, dinv, w2i, w2r, b2)


def _tc3_body(agg_ref, root2_ref, dinv_ref, out_ref):
    agg = jnp.concatenate([agg_ref[0], agg_ref[1]], axis=-1)
    out_ref[...] = jnp.maximum(
        agg * dinv_ref[...][:, None] + root2_ref[...], 0.0)


def _tc3_call(agg, root2, dinv):
    return pl.pallas_call(
        _tc3_body,
        grid=(pl.cdiv(N, _BN),),
        in_specs=[
            pl.BlockSpec((NC, _BN, D_OUT // NC), lambda i: (0, i, 0)),
            pl.BlockSpec((_BN, D_OUT), lambda i: (i, 0)),
            pl.BlockSpec((_BN,), lambda i: (i,)),
        ],
        out_specs=pl.BlockSpec((_BN, D_OUT), lambda i: (i, 0)),
        out_shape=jax.ShapeDtypeStruct((N, D_OUT), jnp.float32),
    )(agg, root2, dinv)


def kernel(x, edge_index, W1_init, W1_root, b1, W2_init, W2_root, b2):
    row = edge_index[0]
    col = edge_index[1]
    rowc = row.reshape(NCH, CHUNK)
    colc = col.reshape(NCH, CHUNK)
    zeros_hid = jnp.zeros((N, D_HID // NC), jnp.float32)
    zeros_out = jnp.zeros((N, D_OUT // NC), jnp.float32)

    deg_parts = _deg_call(col)
    h0s, root1, dinv = _tc1_call(x, W1_init, W1_root, b1, deg_parts)
    agg1 = _agg_hid(h0s, rowc, colc, zeros_hid)
    h1s, root2 = _tc2_call(agg1, root1, dinv, W2_init, W2_root, b2)
    agg2 = _agg_out(h1s, rowc, colc, zeros_out)
    return _tc3_call(agg2, root2, dinv)


# speedup vs baseline: 1.1972x; 1.1972x over previous
import jax
import jax.numpy as jnp
from jax import lax
from jax.experimental import pallas as pl
from jax.experimental.pallas import tpu as pltpu
from jax.experimental.pallas import tpu_sc as plsc

N = 10000
E = 320000
D_IN = 128
D_HID = 128
D_OUT = 64

NC = 2
NS = 16
NW = NC * NS
CHUNK = 128
NCH = E // CHUNK
RPT = 624
TAIL0 = NS * RPT
TAILN = N - TAIL0

_MESH = dict(core_axis_name="c", subcore_axis_name="s")
_SC_PARAMS = pltpu.CompilerParams(
    needs_layout_passes=False, use_tc_tiling_on_sc=False)


DTPW = NCH // NW
DREM = NCH % NW
DMAXT = DTPW + 1


def _deg_body(eic_hbm, out_hbm, idxbuf, acc):
    c = lax.axis_index("c")
    s = lax.axis_index("s")
    w = s * NC + c

    def zero(i, carry):
        acc[pl.ds(i * 16, 16)] = jnp.zeros((16,), jnp.float32)
        return carry

    lax.fori_loop(0, N // 16, zero, 0)

    gstart = w * DTPW + jnp.minimum(w, DREM)
    ndt = jnp.where(w < DREM, DMAXT, DTPW)

    @pl.when(w < DREM)
    def _():
        pltpu.sync_copy(eic_hbm.at[pl.ds(gstart, DMAXT)], idxbuf)

    @pl.when(w >= DREM)
    def _():
        pltpu.sync_copy(eic_hbm.at[pl.ds(gstart, DTPW)],
                        idxbuf.at[pl.ds(0, DTPW)])

    ones = jnp.ones((16,), jnp.float32)

    def body(t, carry):
        for k in range(CHUNK // 16):
            idx = idxbuf[t, 1, pl.ds(k * 16, 16)]
            plsc.addupdate_scatter(acc, [idx], ones)
        return carry

    lax.fori_loop(0, ndt, body, 0)
    pltpu.sync_copy(acc, out_hbm.at[w])


_deg_call = pl.kernel(
    _deg_body,
    out_type=jax.ShapeDtypeStruct((NW, N), jnp.float32),
    mesh=plsc.VectorSubcoreMesh(**_MESH),
    compiler_params=_SC_PARAMS,
    scratch_types=[
        pltpu.VMEM((DMAXT, 2, CHUNK), jnp.int32),
        pltpu.VMEM((N,), jnp.float32),
    ],
)


HALF = NCH // NC
TPW = HALF // NS
TREM = HALF % NS
MAXT = TPW + 1


def _make_agg_body(d, nbuf, wave):
    waves = []
    base = 0
    while base < MAXT:
        waves.append((base, min(wave, MAXT - base), min(wave, TPW - base)))
        base += wave

    def _agg_body(table_hbm, eic_hbm, zeros_hbm, out_hbm,
                  accum, idx, rows, gsems):
        c = lax.axis_index("c")
        s = lax.axis_index("s")

        off = pl.multiple_of(s * RPT, 8)
        pltpu.sync_copy(zeros_hbm.at[pl.ds(off, RPT)],
                        accum.at[pl.ds(off, RPT)])

        @pl.when(s == NS - 1)
        def _():
            pltpu.sync_copy(zeros_hbm.at[pl.ds(TAIL0, TAILN)],
                            accum.at[pl.ds(TAIL0, TAILN)])

        plsc.subcore_barrier()

        gstart = c * HALF + s * TPW + jnp.minimum(s, TREM)
        nt = jnp.where(s < TREM, MAXT, TPW)

        for wbase, wlong, wshort in waves:
            @pl.when(s < TREM)
            def _():
                pltpu.sync_copy(eic_hbm.at[pl.ds(gstart + wbase, wlong)],
                                idx.at[pl.ds(0, wlong)])

            if wshort > 0:
                @pl.when(s >= TREM)
                def _():
                    pltpu.sync_copy(
                        eic_hbm.at[pl.ds(gstart + wbase, wshort)],
                        idx.at[pl.ds(0, wshort)])

            wn = jnp.clip(nt - wbase, 0, wlong)

            for b in range(nbuf):
                @pl.when(b < wn)
                def _():
                    pltpu.async_copy(table_hbm.at[idx.at[b, 0]],
                                     rows.at[b], gsems.at[b])

            def inner(g, carry):
                for b in range(nbuf):
                    ci = g * nbuf + b

                    @pl.when(ci < wn)
                    def _():
                        pltpu.make_async_copy(
                            table_hbm.at[idx.at[b, 0]], rows.at[b],
                            gsems.at[b]).wait()
                        pltpu.sync_copy(rows.at[b],
                                        accum.at[idx.at[ci, 1]], add=True)

                        @pl.when(ci + nbuf < wn)
                        def _():
                            pltpu.async_copy(
                                table_hbm.at[idx.at[ci + nbuf, 0]],
                                rows.at[b], gsems.at[b])

                return carry

            lax.fori_loop(0, (wlong + nbuf - 1) // nbuf, inner, 0)

        plsc.subcore_barrier()
        pltpu.sync_copy(accum.at[pl.ds(off, RPT)],
                        out_hbm.at[c, pl.ds(off, RPT)])

        @pl.when(s == NS - 1)
        def _():
            pltpu.sync_copy(accum.at[pl.ds(TAIL0, TAILN)],
                            out_hbm.at[c, pl.ds(TAIL0, TAILN)])

    return _agg_body


def _make_agg(d, nbuf, wave):
    return pl.kernel(
        _make_agg_body(d, nbuf, wave),
        out_type=jax.ShapeDtypeStruct((NC, N, d), jnp.float32),
        mesh=plsc.VectorSubcoreMesh(**_MESH),
        compiler_params=_SC_PARAMS,
        scratch_types=[
            pltpu.VMEM_SHARED((N, d), jnp.float32),
            pltpu.VMEM((wave, 2, CHUNK), jnp.int32),
            pltpu.VMEM((nbuf, CHUNK, d), jnp.float32),
            pltpu.SemaphoreType.DMA((nbuf,)),
        ],
    )


_agg_hid = _make_agg(D_HID, 2, 40)
_agg_out = _make_agg(D_OUT, 4, MAXT)


def _tc1_body(x_ref, w1i_ref, w1r_ref, b1_ref, degp_ref,
              h0s_ref, root1_ref, dinv_ref):
    x = x_ref[...]
    deg = jnp.sum(degp_ref[...], axis=0)
    dinv = jnp.where(deg > 0, lax.rsqrt(jnp.maximum(deg, 1.0)), 0.0)
    h0 = jnp.dot(x, w1i_ref[...], preferred_element_type=jnp.float32)
    h0s_ref[...] = h0 * dinv[:, None]
    root1_ref[...] = (
        jnp.dot(x, w1r_ref[...], preferred_element_type=jnp.float32)
        + b1_ref[...][None, :]
    )
    dinv_ref[...] = dinv


def _tc1_call(x, w1i, w1r, b1, deg_parts):
    return pl.pallas_call(
        _tc1_body,
        out_shape=[
            jax.ShapeDtypeStruct((N, D_HID), jnp.float32),
            jax.ShapeDtypeStruct((N, D_HID), jnp.float32),
            jax.ShapeDtypeStruct((N,), jnp.float32),
        ],
    )(x, w1i, w1r, b1, deg_parts)


def _tc2_body(agg_ref, root1_ref, dinv_ref, w2i_ref, w2r_ref, b2_ref,
              h1s_ref, root2_ref):
    agg = agg_ref[0] + agg_ref[1]
    dinv = dinv_ref[...]
    out1 = jnp.maximum(agg * dinv[:, None] + root1_ref[...], 0.0)
    h1 = jnp.dot(out1, w2i_ref[...], preferred_element_type=jnp.float32)
    h1s_ref[...] = h1 * dinv[:, None]
    root2_ref[...] = (
        jnp.dot(out1, w2r_ref[...], preferred_element_type=jnp.float32)
        + b2_ref[...][None, :]
    )


def _tc2_call(agg, root1, dinv, w2i, w2r, b2):
    return pl.pallas_call(
        _tc2_body,
        out_shape=[
            jax.ShapeDtypeStruct((N, D_OUT), jnp.float32),
            jax.ShapeDtypeStruct((N, D_OUT), jnp.float32),
        ],
    )(agg, root1, dinv, w2i, w2r, b2)


def _tc3_body(agg_ref, root2_ref, dinv_ref, out_ref):
    agg = agg_ref[0] + agg_ref[1]
    out_ref[...] = jnp.maximum(
        agg * dinv_ref[...][:, None] + root2_ref[...], 0.0)


def _tc3_call(agg, root2, dinv):
    return pl.pallas_call(
        _tc3_body,
        out_shape=jax.ShapeDtypeStruct((N, D_OUT), jnp.float32),
    )(agg, root2, dinv)


def kernel(x, edge_index, W1_init, W1_root, b1, W2_init, W2_root, b2):
    eic = edge_index.reshape(2, NCH, CHUNK).transpose(1, 0, 2)
    zeros_hid = jnp.zeros((N, D_HID), jnp.float32)
    zeros_out = jnp.zeros((N, D_OUT), jnp.float32)

    deg_parts = _deg_call(eic)
    h0s, root1, dinv = _tc1_call(x, W1_init, W1_root, b1, deg_parts)
    agg1 = _agg_hid(h0s, eic, zeros_hid)
    h1s, root2 = _tc2_call(agg1, root1, dinv, W2_init, W2_root, b2)
    agg2 = _agg_out(h1s, eic, zeros_out)
    return _tc3_call(agg2, root2, dinv)


# speedup vs baseline: 1.3772x; 1.1504x over previous
import jax
import jax.numpy as jnp
from jax import lax
from jax.experimental import pallas as pl
from jax.experimental.pallas import tpu as pltpu
from jax.experimental.pallas import tpu_sc as plsc

N = 10000
E = 320000
D_IN = 128
D_HID = 128
D_OUT = 64

NC = 2
NS = 16
NW = NC * NS
CHUNK = 128
NCH = E // CHUNK
RPT = 624
TAIL0 = NS * RPT
TAILN = N - TAIL0

_MESH = dict(core_axis_name="c", subcore_axis_name="s")
_SC_PARAMS = pltpu.CompilerParams(
    needs_layout_passes=False, use_tc_tiling_on_sc=False)


DTPW = NCH // NW
DREM = NCH % NW
DMAXT = DTPW + 1


def _deg_body(eic_hbm, out_hbm, idxbuf, acc):
    c = lax.axis_index("c")
    s = lax.axis_index("s")
    w = s * NC + c

    def zero(i, carry):
        acc[pl.ds(i * 16, 16)] = jnp.zeros((16,), jnp.float32)
        return carry

    lax.fori_loop(0, N // 16, zero, 0)

    gstart = w * DTPW + jnp.minimum(w, DREM)
    ndt = jnp.where(w < DREM, DMAXT, DTPW)

    @pl.when(w < DREM)
    def _():
        pltpu.sync_copy(eic_hbm.at[pl.ds(gstart, DMAXT)], idxbuf)

    @pl.when(w >= DREM)
    def _():
        pltpu.sync_copy(eic_hbm.at[pl.ds(gstart, DTPW)],
                        idxbuf.at[pl.ds(0, DTPW)])

    ones = jnp.ones((16,), jnp.float32)

    def body(t, carry):
        for k in range(CHUNK // 16):
            idx = idxbuf[t, 1, pl.ds(k * 16, 16)]
            plsc.addupdate_scatter(acc, [idx], ones)
        return carry

    lax.fori_loop(0, ndt, body, 0)
    pltpu.sync_copy(acc, out_hbm.at[w])


_deg_call = pl.kernel(
    _deg_body,
    out_type=jax.ShapeDtypeStruct((NW, N), jnp.float32),
    mesh=plsc.VectorSubcoreMesh(**_MESH),
    compiler_params=_SC_PARAMS,
    scratch_types=[
        pltpu.VMEM((DMAXT, 2, CHUNK), jnp.int32),
        pltpu.VMEM((N,), jnp.float32),
    ],
)


HALF = NCH // NC
TPW = HALF // NS
TREM = HALF % NS
MAXT = TPW + 1


def _make_agg_body(d, nbuf, wave, async_scatter=False, interleave_out=False,
                   halfrow_table=False):
    tpw = (NCH if halfrow_table else HALF) // NS
    trem = (NCH if halfrow_table else HALF) % NS
    maxt = tpw + 1
    waves = []
    base = 0
    while base < maxt:
        waves.append((base, min(wave, maxt - base), min(wave, tpw - base)))
        base += wave

    def _agg_body(table_hbm, eic_hbm, zeros_hbm, out_hbm,
                  accum, idx, rows, gsems, ssems):
        c = lax.axis_index("c")
        s = lax.axis_index("s")

        off = pl.multiple_of(s * RPT, 8)
        pltpu.sync_copy(zeros_hbm.at[pl.ds(off, RPT)],
                        accum.at[pl.ds(off, RPT)])

        @pl.when(s == NS - 1)
        def _():
            pltpu.sync_copy(zeros_hbm.at[pl.ds(TAIL0, TAILN)],
                            accum.at[pl.ds(TAIL0, TAILN)])

        plsc.subcore_barrier()

        gstart = (0 if halfrow_table else c * HALF) \
            + s * tpw + jnp.minimum(s, trem)
        nt = jnp.where(s < trem, maxt, tpw)

        for wbase, wlong, wshort in waves:
            @pl.when(s < trem)
            def _():
                pltpu.sync_copy(eic_hbm.at[pl.ds(gstart + wbase, wlong)],
                                idx.at[pl.ds(0, wlong)])

            if wshort > 0:
                @pl.when(s >= trem)
                def _():
                    pltpu.sync_copy(
                        eic_hbm.at[pl.ds(gstart + wbase, wshort)],
                        idx.at[pl.ds(0, wshort)])

            wn = jnp.clip(nt - wbase, 0, wlong)

            if halfrow_table:
                def xform(t, carry):
                    for k in range(CHUNK // 16):
                        sl = pl.ds(k * 16, 16)
                        idx[t, 0, sl] = idx[t, 0, sl] * 2 + c
                    return carry

                lax.fori_loop(0, wn, xform, 0)

            if async_scatter:
                pltpu.async_copy(table_hbm.at[idx.at[0, 0]],
                                 rows.at[0], gsems.at[0])

                def inner(ci, carry):
                    for b in range(nbuf):
                        o = 1 - b

                        @pl.when((ci % nbuf == b) & (ci < wn))
                        def _():
                            pltpu.make_async_copy(
                                table_hbm.at[idx.at[b, 0]], rows.at[b],
                                gsems.at[b]).wait()
                            pltpu.async_copy(rows.at[b],
                                             accum.at[idx.at[ci, 1]],
                                             ssems.at[b], add=True)

                            @pl.when(ci + 1 < wn)
                            def _():
                                @pl.when(ci >= 1)
                                def _():
                                    pltpu.make_async_copy(
                                        rows.at[o],
                                        accum.at[idx.at[ci, 1]],
                                        ssems.at[o]).wait()

                                pltpu.async_copy(
                                    table_hbm.at[idx.at[ci + 1, 0]],
                                    rows.at[o], gsems.at[o])

                    return carry

                lax.fori_loop(0, wlong, inner, 0)

                for b in range(nbuf):
                    @pl.when(b < wn)
                    def _():
                        pltpu.make_async_copy(
                            rows.at[b], accum.at[idx.at[0, 1]],
                            ssems.at[b]).wait()
            else:
                for b in range(nbuf):
                    @pl.when(b < wn)
                    def _():
                        pltpu.async_copy(table_hbm.at[idx.at[b, 0]],
                                         rows.at[b], gsems.at[b])

                def inner(g, carry):
                    for b in range(nbuf):
                        ci = g * nbuf + b

                        @pl.when(ci < wn)
                        def _():
                            pltpu.make_async_copy(
                                table_hbm.at[idx.at[b, 0]], rows.at[b],
                                gsems.at[b]).wait()
                            pltpu.sync_copy(rows.at[b],
                                            accum.at[idx.at[ci, 1]],
                                            add=True)

                            @pl.when(ci + nbuf < wn)
                            def _():
                                pltpu.async_copy(
                                    table_hbm.at[idx.at[ci + nbuf, 0]],
                                    rows.at[b], gsems.at[b])

                    return carry

                lax.fori_loop(0, (wlong + nbuf - 1) // nbuf, inner, 0)

        plsc.subcore_barrier()
        if interleave_out:
            pltpu.sync_copy(accum.at[pl.ds(off, RPT)],
                            out_hbm.at[pl.ds(off, RPT), pl.ds(c * d, d)])

            @pl.when(s == NS - 1)
            def _():
                pltpu.sync_copy(
                    accum.at[pl.ds(TAIL0, TAILN)],
                    out_hbm.at[pl.ds(TAIL0, TAILN), pl.ds(c * d, d)])
        else:
            pltpu.sync_copy(accum.at[pl.ds(off, RPT)],
                            out_hbm.at[c, pl.ds(off, RPT)])

            @pl.when(s == NS - 1)
            def _():
                pltpu.sync_copy(accum.at[pl.ds(TAIL0, TAILN)],
                                out_hbm.at[c, pl.ds(TAIL0, TAILN)])

    return _agg_body


def _make_agg(d, nbuf, wave, async_scatter=False, interleave_out=False,
              halfrow_table=False):
    out_shape = (N, NC * d) if interleave_out else (NC, N, d)
    return pl.kernel(
        _make_agg_body(d, nbuf, wave, async_scatter, interleave_out,
                       halfrow_table),
        out_type=jax.ShapeDtypeStruct(out_shape, jnp.float32),
        mesh=plsc.VectorSubcoreMesh(**_MESH),
        compiler_params=_SC_PARAMS,
        scratch_types=[
            pltpu.VMEM_SHARED((N, d), jnp.float32),
            pltpu.VMEM((wave, 2, CHUNK), jnp.int32),
            pltpu.VMEM((nbuf, CHUNK, d), jnp.float32),
            pltpu.SemaphoreType.DMA((nbuf,)),
            pltpu.SemaphoreType.DMA((nbuf,)),
        ],
    )


_agg_hid = _make_agg(D_HID // NC, 4, NCH // NS + 1,
                     interleave_out=True, halfrow_table=True)
_agg_out = _make_agg(D_OUT, 4, MAXT, interleave_out=True)


def _tc1_body(x_ref, w1i_ref, w1r_ref, b1_ref, degp_ref,
              h0s_ref, root1_ref, dinv_ref):
    x = x_ref[...]
    deg = jnp.sum(degp_ref[...], axis=0)
    dinv = jnp.where(deg > 0, lax.rsqrt(jnp.maximum(deg, 1.0)), 0.0)
    h0 = jnp.dot(x, w1i_ref[...], preferred_element_type=jnp.float32)
    h0s_ref[...] = h0 * dinv[:, None]
    root1_ref[...] = (
        jnp.dot(x, w1r_ref[...], preferred_element_type=jnp.float32)
        + b1_ref[...][None, :]
    )
    dinv_ref[...] = dinv


def _tc1_call(x, w1i, w1r, b1, deg_parts):
    return pl.pallas_call(
        _tc1_body,
        out_shape=[
            jax.ShapeDtypeStruct((N, D_HID), jnp.float32),
            jax.ShapeDtypeStruct((N, D_HID), jnp.float32),
            jax.ShapeDtypeStruct((N,), jnp.float32),
        ],
    )(x, w1i, w1r, b1, deg_parts)


def _tc2_body(agg_ref, root1_ref, dinv_ref, w2i_ref, w2r_ref, b2_ref,
              h1s_ref, root2_ref):
    agg = agg_ref[...]
    dinv = dinv_ref[...]
    out1 = jnp.maximum(agg * dinv[:, None] + root1_ref[...], 0.0)
    h1 = jnp.dot(out1, w2i_ref[...], preferred_element_type=jnp.float32)
    h1s_ref[...] = h1 * dinv[:, None]
    root2_ref[...] = (
        jnp.dot(out1, w2r_ref[...], preferred_element_type=jnp.float32)
        + b2_ref[...][None, :]
    )


def _tc2_call(agg, root1, dinv, w2i, w2r, b2):
    return pl.pallas_call(
        _tc2_body,
        out_shape=[
            jax.ShapeDtypeStruct((N, D_OUT), jnp.float32),
            jax.ShapeDtypeStruct((N, D_OUT), jnp.float32),
        ],
    )(agg, root1, dinv, w2i, w2r, b2)


def _tc3_body(agg_ref, root2_ref, dinv_ref, out_ref):
    agg = agg_ref[:, :D_OUT] + agg_ref[:, D_OUT:]
    out_ref[...] = jnp.maximum(
        agg * dinv_ref[...][:, None] + root2_ref[...], 0.0)


def _tc3_call(agg, root2, dinv):
    return pl.pallas_call(
        _tc3_body,
        out_shape=jax.ShapeDtypeStruct((N, D_OUT), jnp.float32),
    )(agg, root2, dinv)


def kernel(x, edge_index, W1_init, W1_root, b1, W2_init, W2_root, b2):
    eic = edge_index.reshape(2, NCH, CHUNK).transpose(1, 0, 2)
    zeros_hid = jnp.zeros((N, D_HID // NC), jnp.float32)
    zeros_out = jnp.zeros((N, D_OUT), jnp.float32)

    deg_parts = _deg_call(eic)
    h0s, root1, dinv = _tc1_call(x, W1_init, W1_root, b1, deg_parts)
    agg1 = _agg_hid(h0s.reshape(NC * N, D_HID // NC), eic, zeros_hid)
    h1s, root2 = _tc2_call(agg1, root1, dinv, W2_init, W2_root, b2)
    agg2 = _agg_out(h1s, eic, zeros_out)
    return _tc3_call(agg2, root2, dinv)
